# Initial kernel scaffold; baseline (speedup 1.0000x reference)
import jax
import jax.numpy as jnp
from jax.experimental import pallas as pl


def kernel(input_batch, samples, classes):
    raise NotImplementedError("write your pallas kernel here")


# speedup vs baseline: 15.9523x; 15.9523x over previous
import functools

import jax
import jax.numpy as jnp
from jax import lax
from jax.experimental import pallas as pl
from jax.experimental.pallas import tpu as pltpu
from jax.experimental.pallas import tpu_sc as plsc

_NB = 400


def _dist_body(q_ref, s_ref, l2_ref, idx_ref, minv, mini):
    i = pl.program_id(0)

    @pl.when(i == 0)
    def _init():
        minv[...] = jnp.full(minv.shape, jnp.inf, jnp.float32)
        mini[...] = jnp.zeros(mini.shape, jnp.int32)

    q = q_ref[...]
    s = s_ref[...]
    qs = lax.dot_general(q, s, (((1,), (1,)), ((), ())),
                         preferred_element_type=jnp.float32,
                         precision=lax.Precision.HIGHEST)
    sn = lax.dot_general(jnp.ones_like(q), s * s, (((1,), (1,)), ((), ())),
                         preferred_element_type=jnp.float32,
                         precision=lax.Precision.HIGHEST)
    t = sn - 2.0 * qs
    bmin = jnp.min(t, axis=1, keepdims=True)
    barg = jnp.argmin(t, axis=1).astype(jnp.int32)[:, None] + i * _NB
    better = bmin < minv[...]
    minv[...] = jnp.where(better, bmin, minv[...])
    mini[...] = jnp.where(better, barg, mini[...])

    @pl.when(i == pl.num_programs(0) - 1)
    def _fin():
        qn = jnp.sum(q * q, axis=1, keepdims=True)
        l2_ref[...] = jnp.sqrt(jnp.maximum(minv[...] + qn, 0.0))
        idx_ref[...] = mini[...]


def _nearest(b_flat, s_flat):
    bs, d = b_flat.shape
    n = s_flat.shape[0]
    return pl.pallas_call(
        _dist_body,
        grid=(n // _NB,),
        in_specs=[
            pl.BlockSpec((bs, d), lambda i: (0, 0)),
            pl.BlockSpec((_NB, d), lambda i: (i, 0)),
        ],
        out_specs=[
            pl.BlockSpec((bs, 1), lambda i: (0, 0)),
            pl.BlockSpec((bs, 1), lambda i: (0, 0)),
        ],
        out_shape=[
            jax.ShapeDtypeStruct((bs, 1), jnp.float32),
            jax.ShapeDtypeStruct((bs, 1), jnp.int32),
        ],
        scratch_shapes=[
            pltpu.VMEM((bs, 1), jnp.float32),
            pltpu.VMEM((bs, 1), jnp.int32),
        ],
    )(b_flat, s_flat)


def _make_sc_gather(n, d, bs, ncls):
    info = plsc.get_sparse_core_info()
    qpw = 16
    active = bs // qpw

    mesh = plsc.VectorSubcoreMesh(core_axis_name="c", subcore_axis_name="s")

    @functools.partial(
        pl.kernel,
        mesh=mesh,
        compiler_params=pltpu.CompilerParams(needs_layout_passes=False),
        out_type=[
            jax.ShapeDtypeStruct((bs, d), jnp.float32),
            jax.ShapeDtypeStruct((bs * ncls,), jnp.float32),
        ],
        scratch_types=[
            pltpu.VMEM((qpw,), jnp.int32),
            pltpu.VMEM((qpw, d), jnp.float32),
            pltpu.VMEM((n,), jnp.int32),
            pltpu.VMEM((qpw * ncls,), jnp.float32),
            pltpu.SemaphoreType.DMA,
        ],
    )
    def gather(samples_hbm, bidx_hbm, classes_hbm,
               imgs_hbm, pred_hbm, idx_v, rows_v, cls_v, hot_v, sem_r):
        wid = lax.axis_index("s") * info.num_cores + lax.axis_index("c")

        @pl.when(wid < active)
        def _():
            base = wid * qpw
            pltpu.sync_copy(bidx_hbm.at[pl.ds(base, qpw)], idx_v)
            row_dma = pltpu.async_copy(samples_hbm.at[idx_v], rows_v, sem_r)
            pltpu.sync_copy(classes_hbm, cls_v)
            for j in range(ncls):
                hot_v[pl.ds(j * qpw, qpw)] = jnp.zeros((qpw,), jnp.float32)
            idx = idx_v[...]
            cls = plsc.load_gather(cls_v, [idx])
            pos = jnp.arange(qpw, dtype=jnp.int32) * ncls + cls
            plsc.store_scatter(hot_v, [pos], jnp.ones((qpw,), jnp.float32))
            pltpu.sync_copy(hot_v, pred_hbm.at[pl.ds(base * ncls, qpw * ncls)])
            row_dma.wait()
            pltpu.sync_copy(rows_v, imgs_hbm.at[pl.ds(base, qpw)])

    return gather


def kernel(input_batch, samples, classes):
    bs = input_batch.shape[0]
    n = samples.shape[0]
    s_flat = samples.reshape(n, -1)
    b_flat = input_batch.reshape(bs, -1)
    d = s_flat.shape[1]

    l2c, bidx = _nearest(b_flat, s_flat)

    imgs_flat, hot = _make_sc_gather(n, d, bs, 10)(
        s_flat, bidx.reshape(bs), classes)

    pred = hot.reshape(bs, 10)
    imgs = imgs_flat.reshape((bs,) + samples.shape[1:])
    return pred, imgs, l2c.reshape(bs)
